# baseline (speedup 1.0000x reference)
import jax
import jax.numpy as jnp
import numpy as np
from jax import lax
from jax.experimental import pallas as pl
from jax.experimental.pallas import tpu as pltpu
from jax.experimental.pallas import tpu_sc as plsc

H = 64
W = 64
NPAIR = 32
LANES = 16
NCHUNK = W // LANES
BIG = 1e9
NEG = -2e9
ACC0 = 2e9
NACC = 16
NJB = 2


def _lane_iota():
    return lax.iota(jnp.int32, LANES)


def _bcast(x, dtype=jnp.int32):
    return jnp.broadcast_to(jnp.asarray(x, dtype), (LANES,))


def _edge_pass(img_refs, padv, write_fns):
    lanes = _lane_iota()
    padvec = jnp.broadcast_to(padv, (LANES,))
    nimg = len(img_refs)
    row0 = tuple(img_refs[n][0, pl.ds(c0, LANES)]
                 for n in range(nimg) for c0 in range(0, W, LANES))

    def body(r, carry):
        rowm = carry[:nimg * NCHUNK]
        rowc = carry[nimg * NCHUNK:]
        rvec = _bcast(r)
        rp = jnp.minimum(r + 1, H - 1)
        rown = tuple(img_refs[n][rp, pl.ds(c0, LANES)]
                     for n in range(nimg) for c0 in range(0, W, LANES))
        for n in range(nimg):
            img_ref = img_refs[n]
            for ci in range(NCHUNK):
                c0 = ci * LANES
                col = lanes + c0
                cen = rowc[n * NCHUNK + ci]
                up = jnp.where(r == 0, padvec, rowm[n * NCHUNK + ci])
                dn = jnp.where(r == H - 1, padvec, rown[n * NCHUNK + ci])
                if ci == 0:
                    lf = plsc.load_gather(
                        img_ref, [rvec, jnp.maximum(col - 1, 0)])
                    lf = jnp.where(col == 0, padvec, lf)
                else:
                    lf = img_ref[r, pl.ds(c0 - 1, LANES)]
                if ci == NCHUNK - 1:
                    rt = plsc.load_gather(
                        img_ref, [rvec, jnp.minimum(col + 1, W - 1)])
                    rt = jnp.where(col == W - 1, padvec, rt)
                else:
                    rt = img_ref[r, pl.ds(c0 + 1, LANES)]
                mask = (4.0 * cen - (up + dn + lf + rt)) > 0.0
                write_fns[n](r, rvec, c0, col, mask)
        return rowc + rown

    lax.fori_loop(0, H, body, row0 + row0)


def _minplus_block(a_ref, j0, i0f):
    steps = [-2.0 * (i0f + t) for t in range(NACC)]

    def body(k, carry):
        accs, ss, v = carry
        kn = jnp.minimum(k + 1, H - 1)
        vn = tuple(a_ref[kn, pl.ds(j0 + c * LANES, LANES)]
                   for c in range(NJB))
        new_accs = tuple(
            tuple(jnp.minimum(accs[c][t], ss[t] + v[c])
                  for t in range(NACC))
            for c in range(NJB))
        new_ss = tuple(ss[t] + steps[t] for t in range(NACC))
        return (new_accs, new_ss, vn)

    init = (
        tuple(tuple(jnp.full((LANES,), ACC0, jnp.float32)
                    for _ in range(NACC)) for _ in range(NJB)),
        tuple(jnp.zeros((), jnp.float32) for _ in range(NACC)),
        tuple(a_ref[0, pl.ds(j0 + c * LANES, LANES)] for c in range(NJB)),
    )
    accs, _, _ = lax.fori_loop(0, H, body, init)
    return tuple(
        tuple(accs[c][t] + (i0f + t) * (i0f + t) for t in range(NACC))
        for c in range(NJB))


def _sc_kernel(pred_hbm, gt_hbm, out_hbm, img, img2, mt, pm, f, outv, sem):
    w = lax.axis_index("c") * 16 + lax.axis_index("s")
    padv = jnp.where(w % 2 == 0, jnp.float32(1.0), jnp.float32(0.0))
    lanes = _lane_iota()

    pltpu.sync_copy(pred_hbm.at[w], img)
    pltpu.sync_copy(gt_hbm.at[w], img2)

    def write_pred(r, rvec, c0, col, mask):
        pm[r, pl.ds(c0, LANES)] = jnp.where(mask, 0.0, NEG)

    def write_gt(r, rvec, c0, col, mask):
        colf = col.astype(jnp.float32)
        plsc.store_scatter(
            mt, [col, rvec], jnp.where(mask, 0.0, BIG) + colf * colf)

    _edge_pass((img, img2), padv, (write_pred, write_gt))

    for jc in range(0, H, NJB * LANES):
        jcols = [lanes + jc + c * LANES for c in range(NJB)]
        rsqs = []
        for c in range(NJB):
            jcf = jcols[c].astype(jnp.float32)
            rsqs.append(jcf * jcf)
        for i0 in range(0, W, NACC):
            accs = _minplus_block(mt, jc, np.float32(i0))
            for c in range(NJB):
                for t in range(NACC):
                    plsc.store_scatter(
                        f, [jcols[c], _bcast(i0 + t)], accs[c][t] + rsqs[c])

    maxv = jnp.full((LANES,), -1.0, jnp.float32)
    for jc in range(0, W, NJB * LANES):
        for i0 in range(0, H, NACC):
            accs = _minplus_block(f, jc, np.float32(i0))
            for c in range(NJB):
                for t in range(NACC):
                    pmb = pm[i0 + t, pl.ds(jc + c * LANES, LANES)]
                    maxv = jnp.maximum(maxv, accs[c][t] + pmb)

    val = jnp.max(maxv)
    val = jnp.where((val < 0.0) | (val > 1e8), 0.0, val)
    outv[...] = jnp.where(lanes == 0, val, 0.0)
    pltpu.sync_copy(outv, out_hbm.at[w])


@jax.jit
def _hausdorff_sc(predr, gtr):
    mesh = plsc.VectorSubcoreMesh(core_axis_name="c", subcore_axis_name="s")
    run = pl.kernel(
        _sc_kernel,
        out_type=jax.ShapeDtypeStruct((NPAIR, LANES), jnp.float32),
        mesh=mesh,
        scratch_types=[
            pltpu.VMEM((H, W), jnp.float32),
            pltpu.VMEM((H, W), jnp.float32),
            pltpu.VMEM((W, H), jnp.float32),
            pltpu.VMEM((H, W), jnp.float32),
            pltpu.VMEM((H, W), jnp.float32),
            pltpu.VMEM((LANES,), jnp.float32),
            pltpu.SemaphoreType.DMA,
        ],
        compiler_params=pltpu.CompilerParams(needs_layout_passes=False),
    )
    return run(predr, gtr)


def kernel(pred, GT):
    predr = pred.reshape(NPAIR, H, W)
    gtr = GT.reshape(NPAIR, H, W)
    partials = _hausdorff_sc(predr, gtr)
    return (partials.sum() / (H * W)).astype(jnp.float32)


# speedup vs baseline: 1.0112x; 1.0112x over previous
import jax
import jax.numpy as jnp
import numpy as np
from jax import lax
from jax.experimental import pallas as pl
from jax.experimental.pallas import tpu as pltpu
from jax.experimental.pallas import tpu_sc as plsc

H = 64
W = 64
NPAIR = 32
LANES = 16
NCHUNK = W // LANES
BIG = 1e9
NEG = -2e9
ACC0 = 2e9
NACC = 4
NJB = 4


def _lane_iota():
    return lax.iota(jnp.int32, LANES)


def _bcast(x, dtype=jnp.int32):
    return jnp.broadcast_to(jnp.asarray(x, dtype), (LANES,))


def _edge_pass(img_refs, padv, write_fns):
    lanes = _lane_iota()
    padvec = jnp.broadcast_to(padv, (LANES,))
    nimg = len(img_refs)
    row0 = tuple(img_refs[n][0, pl.ds(c0, LANES)]
                 for n in range(nimg) for c0 in range(0, W, LANES))

    def body(r, carry):
        rowm = carry[:nimg * NCHUNK]
        rowc = carry[nimg * NCHUNK:]
        rvec = _bcast(r)
        rp = jnp.minimum(r + 1, H - 1)
        rown = tuple(img_refs[n][rp, pl.ds(c0, LANES)]
                     for n in range(nimg) for c0 in range(0, W, LANES))
        for n in range(nimg):
            img_ref = img_refs[n]
            for ci in range(NCHUNK):
                c0 = ci * LANES
                col = lanes + c0
                cen = rowc[n * NCHUNK + ci]
                up = jnp.where(r == 0, padvec, rowm[n * NCHUNK + ci])
                dn = jnp.where(r == H - 1, padvec, rown[n * NCHUNK + ci])
                if ci == 0:
                    lf = plsc.load_gather(
                        img_ref, [rvec, jnp.maximum(col - 1, 0)])
                    lf = jnp.where(col == 0, padvec, lf)
                else:
                    lf = img_ref[r, pl.ds(c0 - 1, LANES)]
                if ci == NCHUNK - 1:
                    rt = plsc.load_gather(
                        img_ref, [rvec, jnp.minimum(col + 1, W - 1)])
                    rt = jnp.where(col == W - 1, padvec, rt)
                else:
                    rt = img_ref[r, pl.ds(c0 + 1, LANES)]
                mask = (4.0 * cen - (up + dn + lf + rt)) > 0.0
                write_fns[n](r, rvec, c0, col, mask)
        return rowc + rown

    lax.fori_loop(0, H, body, row0 + row0)


def _minplus_block(a_ref, j0, i0f):
    steps = [-2.0 * (i0f + t) for t in range(NACC)]

    def body(k, carry):
        accs, ss, v = carry
        kn = jnp.minimum(k + 1, H - 1)
        vn = tuple(a_ref[kn, pl.ds(j0 + c * LANES, LANES)]
                   for c in range(NJB))
        new_accs = tuple(
            tuple(jnp.minimum(accs[c][t], ss[t] + v[c])
                  for t in range(NACC))
            for c in range(NJB))
        new_ss = tuple(ss[t] + steps[t] for t in range(NACC))
        return (new_accs, new_ss, vn)

    init = (
        tuple(tuple(jnp.full((LANES,), ACC0, jnp.float32)
                    for _ in range(NACC)) for _ in range(NJB)),
        tuple(jnp.zeros((), jnp.float32) for _ in range(NACC)),
        tuple(a_ref[0, pl.ds(j0 + c * LANES, LANES)] for c in range(NJB)),
    )
    accs, _, _ = lax.fori_loop(0, H, body, init)
    return tuple(
        tuple(accs[c][t] + (i0f + t) * (i0f + t) for t in range(NACC))
        for c in range(NJB))


def _sc_kernel(pred_hbm, gt_hbm, out_hbm, img, img2, mt, pm, f, outv, sem):
    w = lax.axis_index("c") * 16 + lax.axis_index("s")
    padv = jnp.where(w % 2 == 0, jnp.float32(1.0), jnp.float32(0.0))
    lanes = _lane_iota()

    pltpu.sync_copy(pred_hbm.at[w], img)
    pltpu.sync_copy(gt_hbm.at[w], img2)

    def write_pred(r, rvec, c0, col, mask):
        pm[r, pl.ds(c0, LANES)] = jnp.where(mask, 0.0, NEG)

    def write_gt(r, rvec, c0, col, mask):
        colf = col.astype(jnp.float32)
        plsc.store_scatter(
            mt, [col, rvec], jnp.where(mask, 0.0, BIG) + colf * colf)

    _edge_pass((img, img2), padv, (write_pred, write_gt))

    for jc in range(0, H, NJB * LANES):
        jcols = [lanes + jc + c * LANES for c in range(NJB)]
        rsqs = []
        for c in range(NJB):
            jcf = jcols[c].astype(jnp.float32)
            rsqs.append(jcf * jcf)
        for i0 in range(0, W, NACC):
            accs = _minplus_block(mt, jc, np.float32(i0))
            for c in range(NJB):
                for t in range(NACC):
                    plsc.store_scatter(
                        f, [jcols[c], _bcast(i0 + t)], accs[c][t] + rsqs[c])

    maxv = jnp.full((LANES,), -1.0, jnp.float32)
    for jc in range(0, W, NJB * LANES):
        for i0 in range(0, H, NACC):
            accs = _minplus_block(f, jc, np.float32(i0))
            for c in range(NJB):
                for t in range(NACC):
                    pmb = pm[i0 + t, pl.ds(jc + c * LANES, LANES)]
                    maxv = jnp.maximum(maxv, accs[c][t] + pmb)

    val = jnp.max(maxv)
    val = jnp.where((val < 0.0) | (val > 1e8), 0.0, val)
    outv[...] = jnp.where(lanes == 0, val, 0.0)
    pltpu.sync_copy(outv, out_hbm.at[w])


@jax.jit
def _hausdorff_sc(predr, gtr):
    mesh = plsc.VectorSubcoreMesh(core_axis_name="c", subcore_axis_name="s")
    run = pl.kernel(
        _sc_kernel,
        out_type=jax.ShapeDtypeStruct((NPAIR, LANES), jnp.float32),
        mesh=mesh,
        scratch_types=[
            pltpu.VMEM((H, W), jnp.float32),
            pltpu.VMEM((H, W), jnp.float32),
            pltpu.VMEM((W, H), jnp.float32),
            pltpu.VMEM((H, W), jnp.float32),
            pltpu.VMEM((H, W), jnp.float32),
            pltpu.VMEM((LANES,), jnp.float32),
            pltpu.SemaphoreType.DMA,
        ],
        compiler_params=pltpu.CompilerParams(needs_layout_passes=False),
    )
    return run(predr, gtr)


def kernel(pred, GT):
    predr = pred.reshape(NPAIR, H, W)
    gtr = GT.reshape(NPAIR, H, W)
    partials = _hausdorff_sc(predr, gtr)
    return (partials.sum() / (H * W)).astype(jnp.float32)


# speedup vs baseline: 1.0274x; 1.0161x over previous
import jax
import jax.numpy as jnp
import numpy as np
from jax import lax
from jax.experimental import pallas as pl
from jax.experimental.pallas import tpu as pltpu
from jax.experimental.pallas import tpu_sc as plsc

H = 64
W = 64
NPAIR = 32
LANES = 16
NCHUNK = W // LANES
BIG = 1e9
NEG = -2e9
ACC0 = 2e9
NACC = 8
NJB = 2


def _lane_iota():
    return lax.iota(jnp.int32, LANES)


def _bcast(x, dtype=jnp.int32):
    return jnp.broadcast_to(jnp.asarray(x, dtype), (LANES,))


def _edge_pass(img_refs, padv, write_fns):
    lanes = _lane_iota()
    padvec = jnp.broadcast_to(padv, (LANES,))
    nimg = len(img_refs)
    row0 = tuple(img_refs[n][0, pl.ds(c0, LANES)]
                 for n in range(nimg) for c0 in range(0, W, LANES))

    def body(r, carry):
        rowm = carry[:nimg * NCHUNK]
        rowc = carry[nimg * NCHUNK:]
        rvec = _bcast(r)
        rp = jnp.minimum(r + 1, H - 1)
        rown = tuple(img_refs[n][rp, pl.ds(c0, LANES)]
                     for n in range(nimg) for c0 in range(0, W, LANES))
        for n in range(nimg):
            img_ref = img_refs[n]
            for ci in range(NCHUNK):
                c0 = ci * LANES
                col = lanes + c0
                cen = rowc[n * NCHUNK + ci]
                up = jnp.where(r == 0, padvec, rowm[n * NCHUNK + ci])
                dn = jnp.where(r == H - 1, padvec, rown[n * NCHUNK + ci])
                if ci == 0:
                    lf = plsc.load_gather(
                        img_ref, [rvec, jnp.maximum(col - 1, 0)])
                    lf = jnp.where(col == 0, padvec, lf)
                else:
                    lf = img_ref[r, pl.ds(c0 - 1, LANES)]
                if ci == NCHUNK - 1:
                    rt = plsc.load_gather(
                        img_ref, [rvec, jnp.minimum(col + 1, W - 1)])
                    rt = jnp.where(col == W - 1, padvec, rt)
                else:
                    rt = img_ref[r, pl.ds(c0 + 1, LANES)]
                mask = (4.0 * cen - (up + dn + lf + rt)) > 0.0
                write_fns[n](r, rvec, c0, col, mask)
        return rowc + rown

    lax.fori_loop(0, H, body, row0 + row0)


def _minplus_block(a_ref, j0, i0f):
    steps = [-2.0 * (i0f + t) for t in range(NACC)]

    def body(k, carry):
        accs, ss, v = carry
        kn = jnp.minimum(k + 1, H - 1)
        vn = tuple(a_ref[kn, pl.ds(j0 + c * LANES, LANES)]
                   for c in range(NJB))
        new_accs = tuple(
            tuple(jnp.minimum(accs[c][t], ss[t] + v[c])
                  for t in range(NACC))
            for c in range(NJB))
        new_ss = tuple(ss[t] + steps[t] for t in range(NACC))
        return (new_accs, new_ss, vn)

    init = (
        tuple(tuple(jnp.full((LANES,), ACC0, jnp.float32)
                    for _ in range(NACC)) for _ in range(NJB)),
        tuple(jnp.zeros((), jnp.float32) for _ in range(NACC)),
        tuple(a_ref[0, pl.ds(j0 + c * LANES, LANES)] for c in range(NJB)),
    )
    accs, _, _ = lax.fori_loop(0, H, body, init)
    return tuple(
        tuple(accs[c][t] + (i0f + t) * (i0f + t) for t in range(NACC))
        for c in range(NJB))


def _sc_kernel(pred_hbm, gt_hbm, out_hbm, img, img2, mt, pm, f, outv,
               sem, sem2):
    w = lax.axis_index("c") * 16 + lax.axis_index("s")
    padv = jnp.where(w % 2 == 0, jnp.float32(1.0), jnp.float32(0.0))
    lanes = _lane_iota()

    cp1 = pltpu.make_async_copy(pred_hbm.at[w], img, sem)
    cp2 = pltpu.make_async_copy(gt_hbm.at[w], img2, sem2)
    cp1.start()
    cp2.start()
    cp1.wait()
    cp2.wait()

    def write_pred(r, rvec, c0, col, mask):
        pm[r, pl.ds(c0, LANES)] = jnp.where(mask, 0.0, NEG)

    def write_gt(r, rvec, c0, col, mask):
        colf = col.astype(jnp.float32)
        plsc.store_scatter(
            mt, [col, rvec], jnp.where(mask, 0.0, BIG) + colf * colf)

    _edge_pass((img, img2), padv, (write_pred, write_gt))

    for jc in range(0, H, NJB * LANES):
        jcols = [lanes + jc + c * LANES for c in range(NJB)]
        rsqs = []
        for c in range(NJB):
            jcf = jcols[c].astype(jnp.float32)
            rsqs.append(jcf * jcf)
        for i0 in range(0, W, NACC):
            accs = _minplus_block(mt, jc, np.float32(i0))
            for c in range(NJB):
                for t in range(NACC):
                    plsc.store_scatter(
                        f, [jcols[c], _bcast(i0 + t)], accs[c][t] + rsqs[c])

    maxv = jnp.full((LANES,), -1.0, jnp.float32)
    for jc in range(0, W, NJB * LANES):
        for i0 in range(0, H, NACC):
            accs = _minplus_block(f, jc, np.float32(i0))
            for c in range(NJB):
                for t in range(NACC):
                    pmb = pm[i0 + t, pl.ds(jc + c * LANES, LANES)]
                    maxv = jnp.maximum(maxv, accs[c][t] + pmb)

    val = jnp.max(maxv)
    val = jnp.where((val < 0.0) | (val > 1e8), 0.0, val)
    outv[...] = jnp.where(lanes == 0, val, 0.0)
    pltpu.sync_copy(outv, out_hbm.at[w])


@jax.jit
def _hausdorff_sc(predr, gtr):
    mesh = plsc.VectorSubcoreMesh(core_axis_name="c", subcore_axis_name="s")
    run = pl.kernel(
        _sc_kernel,
        out_type=jax.ShapeDtypeStruct((NPAIR, LANES), jnp.float32),
        mesh=mesh,
        scratch_types=[
            pltpu.VMEM((H, W), jnp.float32),
            pltpu.VMEM((H, W), jnp.float32),
            pltpu.VMEM((W, H), jnp.float32),
            pltpu.VMEM((H, W), jnp.float32),
            pltpu.VMEM((H, W), jnp.float32),
            pltpu.VMEM((LANES,), jnp.float32),
            pltpu.SemaphoreType.DMA,
            pltpu.SemaphoreType.DMA,
        ],
        compiler_params=pltpu.CompilerParams(needs_layout_passes=False),
    )
    return run(predr, gtr)


def kernel(pred, GT):
    predr = pred.reshape(NPAIR, H, W)
    gtr = GT.reshape(NPAIR, H, W)
    partials = _hausdorff_sc(predr, gtr)
    return (partials.sum() / (H * W)).astype(jnp.float32)


# speedup vs baseline: 1.0320x; 1.0045x over previous
import jax
import jax.numpy as jnp
import numpy as np
from jax import lax
from jax.experimental import pallas as pl
from jax.experimental.pallas import tpu as pltpu
from jax.experimental.pallas import tpu_sc as plsc

H = 64
W = 64
NPAIR = 32
LANES = 16
NCHUNK = W // LANES
BIG = 1e9
NEG = -2e9
ACC0 = 2e9
NACC = 8
NJB = 2


def _lane_iota():
    return lax.iota(jnp.int32, LANES)


def _bcast(x, dtype=jnp.int32):
    return jnp.broadcast_to(jnp.asarray(x, dtype), (LANES,))


def _edge_pass(img_refs, padv, write_fns):
    lanes = _lane_iota()
    padvec = jnp.broadcast_to(padv, (LANES,))
    nimg = len(img_refs)
    row0 = tuple(img_refs[n][0, pl.ds(c0, LANES)]
                 for n in range(nimg) for c0 in range(0, W, LANES))

    def body(r, carry):
        rowm = carry[:nimg * NCHUNK]
        rowc = carry[nimg * NCHUNK:]
        rvec = _bcast(r)
        rp = jnp.minimum(r + 1, H - 1)
        rown = tuple(img_refs[n][rp, pl.ds(c0, LANES)]
                     for n in range(nimg) for c0 in range(0, W, LANES))
        for n in range(nimg):
            img_ref = img_refs[n]
            for ci in range(NCHUNK):
                c0 = ci * LANES
                col = lanes + c0
                cen = rowc[n * NCHUNK + ci]
                up = jnp.where(r == 0, padvec, rowm[n * NCHUNK + ci])
                dn = jnp.where(r == H - 1, padvec, rown[n * NCHUNK + ci])
                if ci == 0:
                    lf = plsc.load_gather(
                        img_ref, [rvec, jnp.maximum(col - 1, 0)])
                    lf = jnp.where(col == 0, padvec, lf)
                else:
                    lf = img_ref[r, pl.ds(c0 - 1, LANES)]
                if ci == NCHUNK - 1:
                    rt = plsc.load_gather(
                        img_ref, [rvec, jnp.minimum(col + 1, W - 1)])
                    rt = jnp.where(col == W - 1, padvec, rt)
                else:
                    rt = img_ref[r, pl.ds(c0 + 1, LANES)]
                mask = (4.0 * cen - (up + dn + lf + rt)) > 0.0
                write_fns[n](r, rvec, c0, col, mask)
        return rowc + rown

    lax.fori_loop(0, H, body, row0 + row0)


def _minplus_block(a_ref, j0, i0f):
    steps = [-2.0 * (i0f + t) for t in range(NACC)]

    def body(k, carry):
        accs, ss, v = carry
        vn = tuple(a_ref[k + 1, pl.ds(j0 + c * LANES, LANES)]
                   for c in range(NJB))
        new_accs = tuple(
            tuple(jnp.minimum(accs[c][t], ss[t] + v[c])
                  for t in range(NACC))
            for c in range(NJB))
        new_ss = tuple(ss[t] + steps[t] for t in range(NACC))
        return (new_accs, new_ss, vn)

    init = (
        tuple(tuple(jnp.full((LANES,), ACC0, jnp.float32)
                    for _ in range(NACC)) for _ in range(NJB)),
        tuple(jnp.zeros((), jnp.float32) for _ in range(NACC)),
        tuple(a_ref[0, pl.ds(j0 + c * LANES, LANES)] for c in range(NJB)),
    )
    accs, _, _ = lax.fori_loop(0, H, body, init)
    return tuple(
        tuple(accs[c][t] + (i0f + t) * (i0f + t) for t in range(NACC))
        for c in range(NJB))


def _sc_kernel(pred_hbm, gt_hbm, out_hbm, img, img2, mt, pm, f, outv,
               sem, sem2):
    w = lax.axis_index("c") * 16 + lax.axis_index("s")
    padv = jnp.where(w % 2 == 0, jnp.float32(1.0), jnp.float32(0.0))
    lanes = _lane_iota()

    cp1 = pltpu.make_async_copy(pred_hbm.at[w], img, sem)
    cp2 = pltpu.make_async_copy(gt_hbm.at[w], img2, sem2)
    cp1.start()
    cp2.start()
    cp1.wait()
    cp2.wait()

    def write_pred(r, rvec, c0, col, mask):
        pm[r, pl.ds(c0, LANES)] = jnp.where(mask, 0.0, NEG)

    def write_gt(r, rvec, c0, col, mask):
        colf = col.astype(jnp.float32)
        plsc.store_scatter(
            mt, [col, rvec], jnp.where(mask, 0.0, BIG) + colf * colf)

    _edge_pass((img, img2), padv, (write_pred, write_gt))

    for jc in range(0, H, NJB * LANES):
        jcols = [lanes + jc + c * LANES for c in range(NJB)]
        rsqs = []
        for c in range(NJB):
            jcf = jcols[c].astype(jnp.float32)
            rsqs.append(jcf * jcf)
        for i0 in range(0, W, NACC):
            accs = _minplus_block(mt, jc, np.float32(i0))
            for c in range(NJB):
                for t in range(NACC):
                    plsc.store_scatter(
                        f, [jcols[c], _bcast(i0 + t)], accs[c][t] + rsqs[c])

    maxv = jnp.full((LANES,), -1.0, jnp.float32)
    for jc in range(0, W, NJB * LANES):
        for i0 in range(0, H, NACC):
            accs = _minplus_block(f, jc, np.float32(i0))
            for c in range(NJB):
                for t in range(NACC):
                    pmb = pm[i0 + t, pl.ds(jc + c * LANES, LANES)]
                    maxv = jnp.maximum(maxv, accs[c][t] + pmb)

    val = jnp.max(maxv)
    val = jnp.where((val < 0.0) | (val > 1e8), 0.0, val)
    outv[...] = jnp.where(lanes == 0, val, 0.0)
    pltpu.sync_copy(outv, out_hbm.at[w])


@jax.jit
def _hausdorff_sc(predr, gtr):
    mesh = plsc.VectorSubcoreMesh(core_axis_name="c", subcore_axis_name="s")
    run = pl.kernel(
        _sc_kernel,
        out_type=jax.ShapeDtypeStruct((NPAIR, LANES), jnp.float32),
        mesh=mesh,
        scratch_types=[
            pltpu.VMEM((H, W), jnp.float32),
            pltpu.VMEM((H, W), jnp.float32),
            pltpu.VMEM((W + 1, H), jnp.float32),
            pltpu.VMEM((H, W), jnp.float32),
            pltpu.VMEM((H + 1, W), jnp.float32),
            pltpu.VMEM((LANES,), jnp.float32),
            pltpu.SemaphoreType.DMA,
            pltpu.SemaphoreType.DMA,
        ],
        compiler_params=pltpu.CompilerParams(needs_layout_passes=False),
    )
    return run(predr, gtr)


def kernel(pred, GT):
    predr = pred.reshape(NPAIR, H, W)
    gtr = GT.reshape(NPAIR, H, W)
    partials = _hausdorff_sc(predr, gtr)
    return (partials.sum() / (H * W)).astype(jnp.float32)
